# baseline (speedup 1.0000x reference)
import numpy as np

import jax
import jax.numpy as jnp
from jax.experimental import pallas as pl
from jax.experimental.pallas import tpu as pltpu

_CP = 128


def _round_up(x, m):
  return ((x + m - 1) // m) * m


def _body(dims, x2_ref, w1_ref, w2_ref, w3_ref, sh_ref, o_ref,
          xcat2, xcat3):
  B, PR, QR, OP = dims
  R = B * PR
  sh1 = sh_ref[0:1, :]
  sh2 = sh_ref[1:2, :]
  sh3 = sh_ref[2:3, :]

  x2 = x2_ref[...].reshape(R, x2_ref.shape[2])
  for u in range(4):
    ye = jnp.dot(x2, w1_ref[2 * u], preferred_element_type=jnp.float32)
    yo = jnp.dot(x2, w1_ref[2 * u + 1], preferred_element_type=jnp.float32)
    p = jnp.maximum(jnp.maximum(ye, yo) + sh1, 0.0).astype(jnp.bfloat16)
    for q in range(3):
      if u + 4 * q < 11:
        xcat2[0:R - q, _CP * (u + 4 * q):_CP * (u + 4 * q + 1)] = p[q:R]

  for u in range(2):
    aa = jnp.dot(xcat2[0:R, _CP * (2 * u):_CP * (2 * u) + 8 * _CP],
                 w2_ref[...], preferred_element_type=jnp.float32)
    ab = jnp.dot(xcat2[0:R, _CP * (2 * u + 1):_CP * (2 * u + 1) + 8 * _CP],
                 w2_ref[...], preferred_element_type=jnp.float32)
    qv = jnp.maximum(jnp.maximum(aa, ab) + sh2, 0.0).astype(jnp.bfloat16)
    for q in range(3):
      if u + 2 * q < 5:
        xcat3[0:R - q, _CP * (u + 2 * q):_CP * (u + 2 * q + 1)] = qv[q:R]

  accs3 = [jnp.dot(xcat3[0:R, _CP * t:_CP * t + 4 * _CP], w3_ref[...],
                   preferred_element_type=jnp.float32) for t in range(2)]
  fin = jnp.maximum(jnp.maximum(accs3[0], accs3[1]) + sh3, 0.0)
  fin_bf = fin.astype(jnp.bfloat16)
  for b in range(B):
    ft = jnp.transpose(fin_bf[b * PR:b * PR + OP], (1, 0))
    o_ref[b] = ft[0:64, :]


@jax.jit
def kernel(x_ncl, w1, w2, w3, shifts):
  N, c_in, L = x_ncl.shape
  KC = w1.shape[0]
  K1 = KC // c_in
  K2, K3 = w2.shape[0], w3.shape[0]

  L0 = L + 8
  L_out1 = L0 - K1 + 1
  L_p1 = L_out1 // 2
  L_out2 = L_p1 - K2 + 1
  L_p2 = L_out2 // 2
  L_out3 = L_p2 - K3 + 1
  L_p3 = L_out3 // 2

  OP = _round_up(L_p3, 8)
  QR = OP + 8
  PR = QR + 8

  x = jnp.pad(x_ncl, ((0, 0), (0, 0), (4, 4)))
  xb = x.reshape(N, c_in, L0 // K1, K1)
  xb = jnp.pad(xb, ((0, 0), (0, 0), (0, PR + 3 - L0 // K1), (0, 0)))
  xb = jnp.transpose(xb, (0, 2, 1, 3)).reshape(N, PR + 3, KC)
  x2 = jnp.concatenate([xb[:, i:PR + i] for i in range(4)], axis=2)
  x2 = x2.astype(jnp.bfloat16)

  j = np.arange(4 * KC)
  b, c, d = j // KC, (j % KC) // K1, j % K1
  W1_np = []
  for o in range(K1):
    idx = K1 * b + d - o
    valid = (idx >= 0) & (idx < K1)
    rows = np.clip(c_in * idx + c, 0, KC - 1)
    W1_np.append((rows, valid))
  W1 = jnp.stack([jnp.where(jnp.asarray(v)[:, None], w1[jnp.asarray(r)], 0.0)
                  for r, v in W1_np]).astype(jnp.bfloat16)
  W2 = w2.reshape(K2 * _CP, _CP).astype(jnp.bfloat16)
  W3 = w3.reshape(K3 * _CP, _CP).astype(jnp.bfloat16)

  B = 4 if N % 4 == 0 else 1
  out = pl.pallas_call(
      lambda *refs: _body((B, PR, QR, OP), *refs),
      out_shape=jax.ShapeDtypeStruct((N, 64, OP), jnp.bfloat16),
      grid=(N // B,),
      in_specs=[
          pl.BlockSpec((B, PR, 4 * KC), lambda n: (n, 0, 0)),
          pl.BlockSpec(W1.shape, lambda n: (0, 0, 0)),
          pl.BlockSpec(W2.shape, lambda n: (0, 0)),
          pl.BlockSpec(W3.shape, lambda n: (0, 0)),
          pl.BlockSpec(shifts.shape, lambda n: (0, 0)),
      ],
      out_specs=pl.BlockSpec((B, 64, OP), lambda n: (n, 0, 0)),
      scratch_shapes=[
          pltpu.VMEM((B * PR, 12 * _CP), jnp.bfloat16),
          pltpu.VMEM((B * PR, 6 * _CP), jnp.bfloat16),
      ],
      compiler_params=pltpu.CompilerParams(
          dimension_semantics=("parallel",)),
  )(x2, W1, W2, W3, shifts)

  return out[:, :50, :L_p3].astype(jnp.float32)


# speedup vs baseline: 1.2628x; 1.2628x over previous
import numpy as np

import jax
import jax.numpy as jnp
from jax.experimental import pallas as pl
from jax.experimental.pallas import tpu as pltpu

_CP = 128


def _round_up(x, m):
  return ((x + m - 1) // m) * m


def _body(dims, xv_ref, w1_ref, w2_ref, w3_ref, sh_ref, o_ref,
          xcat2, xcat3):
  B, BS, OP = dims
  RT = B * BS
  RW = RT - 8
  sh1 = sh_ref[0:1, :]
  sh2 = sh_ref[1:2, :]
  sh3 = sh_ref[2:3, :]

  xv = xv_ref[...].reshape(RT, xv_ref.shape[2])
  x2 = jnp.concatenate([xv[0:RW], xv[1:RW + 1]], axis=1)

  for u in range(4):
    ye = jnp.dot(x2, w1_ref[2 * u], preferred_element_type=jnp.float32)
    yo = jnp.dot(x2, w1_ref[2 * u + 1], preferred_element_type=jnp.float32)
    p = jnp.maximum(jnp.maximum(ye, yo) + sh1, 0.0).astype(jnp.bfloat16)
    for q in range(3):
      if u + 4 * q < 11:
        xcat2[0:RW - q, _CP * (u + 4 * q):_CP * (u + 4 * q + 1)] = p[q:RW]

  for u in range(2):
    aa = jnp.dot(xcat2[0:RW, _CP * (2 * u):_CP * (2 * u) + 8 * _CP],
                 w2_ref[...], preferred_element_type=jnp.float32)
    ab = jnp.dot(xcat2[0:RW, _CP * (2 * u + 1):_CP * (2 * u + 1) + 8 * _CP],
                 w2_ref[...], preferred_element_type=jnp.float32)
    qv = jnp.maximum(jnp.maximum(aa, ab) + sh2, 0.0).astype(jnp.bfloat16)
    for q in range(3):
      if u + 2 * q < 5:
        xcat3[0:RW - q, _CP * (u + 2 * q):_CP * (u + 2 * q + 1)] = qv[q:RW]

  accs3 = [jnp.dot(xcat3[0:RW, _CP * t:_CP * t + 4 * _CP], w3_ref[...],
                   preferred_element_type=jnp.float32) for t in range(2)]
  fin = jnp.maximum(jnp.maximum(accs3[0], accs3[1]) + sh3, 0.0)
  fin_bf = fin.astype(jnp.bfloat16)
  for b in range(B):
    ft = jnp.transpose(fin_bf[b * BS:b * BS + OP], (1, 0))
    o_ref[b] = ft[0:64, :]


@jax.jit
def kernel(x_ncl, w1, w2, w3, shifts):
  N, c_in, L = x_ncl.shape
  KC = w1.shape[0]
  K1 = KC // c_in
  K2, K3 = w2.shape[0], w3.shape[0]

  L0 = L + 8
  L_out1 = L0 - K1 + 1
  L_p1 = L_out1 // 2
  L_out2 = L_p1 - K2 + 1
  L_p2 = L_out2 // 2
  L_out3 = L_p2 - K3 + 1
  L_p3 = L_out3 // 2

  OP = _round_up(L_p3, 8)
  BS = OP + 24

  xp = jnp.pad(x_ncl, ((0, 0), (0, 0), (4, K1 * BS - 4 - L)))
  xb = xp.astype(jnp.bfloat16).reshape(N, c_in, BS, K1)
  xv = jnp.transpose(xb, (0, 2, 1, 3)).reshape(N, BS, KC)

  j = np.arange(2 * KC)
  b, c, d = j // KC, (j % KC) // K1, j % K1
  W1_np = []
  for o in range(K1):
    idx = K1 * b + d - o
    valid = (idx >= 0) & (idx < K1)
    rows = np.clip(c_in * idx + c, 0, KC - 1)
    W1_np.append((rows, valid))
  W1 = jnp.stack([jnp.where(jnp.asarray(v)[:, None], w1[jnp.asarray(r)], 0.0)
                  for r, v in W1_np]).astype(jnp.bfloat16)
  W2 = w2.reshape(K2 * _CP, _CP).astype(jnp.bfloat16)
  W3 = w3.reshape(K3 * _CP, _CP).astype(jnp.bfloat16)

  B = 4 if N % 4 == 0 else 1
  RW = B * BS - 8
  out = pl.pallas_call(
      lambda *refs: _body((B, BS, OP), *refs),
      out_shape=jax.ShapeDtypeStruct((N, 64, OP), jnp.bfloat16),
      grid=(N // B,),
      in_specs=[
          pl.BlockSpec((B, BS, KC), lambda n: (n, 0, 0)),
          pl.BlockSpec(W1.shape, lambda n: (0, 0, 0)),
          pl.BlockSpec(W2.shape, lambda n: (0, 0)),
          pl.BlockSpec(W3.shape, lambda n: (0, 0)),
          pl.BlockSpec(shifts.shape, lambda n: (0, 0)),
      ],
      out_specs=pl.BlockSpec((B, 64, OP), lambda n: (n, 0, 0)),
      scratch_shapes=[
          pltpu.VMEM((RW, 12 * _CP), jnp.bfloat16),
          pltpu.VMEM((RW, 6 * _CP), jnp.bfloat16),
      ],
      compiler_params=pltpu.CompilerParams(
          dimension_semantics=("parallel",)),
  )(xv, W1, W2, W3, shifts)

  return out[:, :50, :L_p3].astype(jnp.float32)


# speedup vs baseline: 1.3915x; 1.1019x over previous
import numpy as np

import jax
import jax.numpy as jnp
from jax.experimental import pallas as pl
from jax.experimental.pallas import tpu as pltpu

_CP = 128


def _round_up(x, m):
  return ((x + m - 1) // m) * m


def _body(dims, xv_ref, w1_ref, w2_ref, w3_ref, sh_ref, o_ref,
          xcat2, xcat3):
  B, BS, OP = dims
  RT = B * BS
  RW = RT - 8
  sh1 = sh_ref[0:1, :]
  sh2 = sh_ref[1:2, :]
  sh3 = sh_ref[2:3, :]

  xv = xv_ref[...].reshape(RT, xv_ref.shape[2])
  x2 = jnp.concatenate([xv[0:RW], xv[1:RW + 1]], axis=1)

  yall = jnp.dot(x2, w1_ref[...], preferred_element_type=jnp.float32)
  for u in range(4):
    p = jnp.maximum(
        jnp.maximum(yall[:, 2 * _CP * u:2 * _CP * u + _CP],
                    yall[:, 2 * _CP * u + _CP:2 * _CP * u + 2 * _CP]) + sh1,
        0.0).astype(jnp.bfloat16)
    for q in range(3):
      if u + 4 * q < 11:
        xcat2[0:RW - q, _CP * (u + 4 * q):_CP * (u + 4 * q + 1)] = p[q:RW]

  for u in range(2):
    acc = jnp.dot(xcat2[0:RW, 2 * _CP * u:2 * _CP * u + 9 * _CP],
                  w2_ref[...], preferred_element_type=jnp.float32)
    qv = jnp.maximum(
        jnp.maximum(acc[:, 0:_CP], acc[:, _CP:2 * _CP]) + sh2,
        0.0).astype(jnp.bfloat16)
    for q in range(3):
      if u + 2 * q < 5:
        xcat3[0:RW - q, _CP * (u + 2 * q):_CP * (u + 2 * q + 1)] = qv[q:RW]

  acc3 = jnp.dot(xcat3[0:RW, 0:5 * _CP], w3_ref[...],
                 preferred_element_type=jnp.float32)
  fin = jnp.maximum(
      jnp.maximum(acc3[:, 0:_CP], acc3[:, _CP:2 * _CP]) + sh3, 0.0)
  fin_bf = fin.astype(jnp.bfloat16)
  for b in range(B):
    ft = jnp.transpose(fin_bf[b * BS:b * BS + OP], (1, 0))
    o_ref[b] = ft[0:64, :]


@jax.jit
def kernel(x_ncl, w1, w2, w3, shifts):
  N, c_in, L = x_ncl.shape
  KC = w1.shape[0]
  K1 = KC // c_in
  K2, K3 = w2.shape[0], w3.shape[0]

  L0 = L + 8
  L_out1 = L0 - K1 + 1
  L_p1 = L_out1 // 2
  L_out2 = L_p1 - K2 + 1
  L_p2 = L_out2 // 2
  L_out3 = L_p2 - K3 + 1
  L_p3 = L_out3 // 2

  OP = _round_up(L_p3, 8)
  BS = OP + 24

  xp = jnp.pad(x_ncl, ((0, 0), (0, 0), (4, K1 * BS - 4 - L)))
  xb = xp.astype(jnp.bfloat16).reshape(N, c_in, BS, K1)
  xv = jnp.transpose(xb, (0, 2, 1, 3)).reshape(N, BS, KC)

  j = np.arange(2 * KC)
  b, c, d = j // KC, (j % KC) // K1, j % K1
  W1_np = []
  for o in range(K1):
    idx = K1 * b + d - o
    valid = (idx >= 0) & (idx < K1)
    rows = np.clip(c_in * idx + c, 0, KC - 1)
    W1_np.append((rows, valid))
  W1 = jnp.concatenate(
      [jnp.where(jnp.asarray(v)[:, None], w1[jnp.asarray(r)], 0.0)
       for r, v in W1_np], axis=1).astype(jnp.bfloat16)
  W2h = w2.reshape(K2 * _CP, _CP)
  W2 = jnp.concatenate([jnp.pad(W2h, ((0, _CP), (0, 0))),
                        jnp.pad(W2h, ((_CP, 0), (0, 0)))],
                       axis=1).astype(jnp.bfloat16)
  W3h = w3.reshape(K3 * _CP, _CP)
  W3 = jnp.concatenate([jnp.pad(W3h, ((0, _CP), (0, 0))),
                        jnp.pad(W3h, ((_CP, 0), (0, 0)))],
                       axis=1).astype(jnp.bfloat16)

  B = 4 if N % 4 == 0 else 1
  RW = B * BS - 8
  out = pl.pallas_call(
      lambda *refs: _body((B, BS, OP), *refs),
      out_shape=jax.ShapeDtypeStruct((N, 64, OP), jnp.bfloat16),
      grid=(N // B,),
      in_specs=[
          pl.BlockSpec((B, BS, KC), lambda n: (n, 0, 0)),
          pl.BlockSpec(W1.shape, lambda n: (0, 0)),
          pl.BlockSpec(W2.shape, lambda n: (0, 0)),
          pl.BlockSpec(W3.shape, lambda n: (0, 0)),
          pl.BlockSpec(shifts.shape, lambda n: (0, 0)),
      ],
      out_specs=pl.BlockSpec((B, 64, OP), lambda n: (n, 0, 0)),
      scratch_shapes=[
          pltpu.VMEM((RW, 12 * _CP), jnp.bfloat16),
          pltpu.VMEM((RW, 6 * _CP), jnp.bfloat16),
      ],
      compiler_params=pltpu.CompilerParams(
          dimension_semantics=("parallel",)),
  )(xv, W1, W2, W3, shifts)

  return out[:, :50, :L_p3].astype(jnp.float32)


# speedup vs baseline: 1.5520x; 1.1153x over previous
import numpy as np

import jax
import jax.numpy as jnp
from jax.experimental import pallas as pl
from jax.experimental.pallas import tpu as pltpu

_CP = 128


def _round_up(x, m):
  return ((x + m - 1) // m) * m


def _body(dims, s2_plan, s3_plan, xv_ref, w1_ref, sh_ref, *rest):
  B, BS, OP = dims
  n2, n3 = len(s2_plan[0]) + len(s2_plan[1]), len(s3_plan)
  w2_refs = rest[:n2]
  w3_refs = rest[n2:n2 + n3]
  o_ref, pb, qb = rest[n2 + n3:]
  sh1 = sh_ref[0:1, :]
  sh2 = sh_ref[1:2, :]
  sh3 = sh_ref[2:3, :]

  for b in range(B):
    xvb = xv_ref[b]
    x2b = jnp.concatenate([xvb[0:BS], xvb[1:BS + 1]], axis=1)

    for u in range(4):
      y = jnp.dot(x2b, w1_ref[:, 2 * _CP * u:2 * _CP * (u + 1)],
                  preferred_element_type=jnp.float32)
      p = jnp.maximum(
          jnp.maximum(y[:, 0:_CP], y[:, _CP:2 * _CP]) + sh1,
          0.0).astype(jnp.bfloat16)
      pb[0:BS, _CP * u:_CP * (u + 1)] = p

    wi = 0
    for u in range(2):
      acc = None
      for q, t0, t1 in s2_plan[u]:
        d = jnp.dot(pb[q:q + BS, _CP * t0:_CP * t1], w2_refs[wi][...],
                    preferred_element_type=jnp.float32)
        acc = d if acc is None else acc + d
        wi += 1
      qv = jnp.maximum(
          jnp.maximum(acc[:, 0:_CP], acc[:, _CP:2 * _CP]) + sh2,
          0.0).astype(jnp.bfloat16)
      qb[0:BS, _CP * u:_CP * (u + 1)] = qv

    acc3 = None
    for wi3, (q, t0, t1) in enumerate(s3_plan):
      d = jnp.dot(qb[q:q + BS, _CP * t0:_CP * t1], w3_refs[wi3][...],
                  preferred_element_type=jnp.float32)
      acc3 = d if acc3 is None else acc3 + d
    fin = jnp.maximum(
        jnp.maximum(acc3[:, 0:_CP], acc3[:, _CP:2 * _CP]) + sh3, 0.0)
    fin_bf = fin.astype(jnp.bfloat16)
    ft = jnp.transpose(fin_bf[0:OP], (1, 0))
    o_ref[b] = ft[0:64, :]


@jax.jit
def kernel(x_ncl, w1, w2, w3, shifts):
  N, c_in, L = x_ncl.shape
  KC = w1.shape[0]
  K1 = KC // c_in
  K2, K3 = w2.shape[0], w3.shape[0]

  L0 = L + 8
  L_out1 = L0 - K1 + 1
  L_p1 = L_out1 // 2
  L_out2 = L_p1 - K2 + 1
  L_p2 = L_out2 // 2
  L_out3 = L_p2 - K3 + 1
  L_p3 = L_out3 // 2

  OP = _round_up(L_p3, 8)
  BS = OP + 24
  BSX = BS + 8

  xp = jnp.pad(x_ncl, ((0, 0), (0, 0), (4, K1 * BSX - 4 - L)))
  xb = xp.astype(jnp.bfloat16).reshape(N, c_in, BSX, K1)
  xv = jnp.transpose(xb, (0, 2, 1, 3)).reshape(N, BSX, KC)

  j = np.arange(2 * KC)
  b, c, d = j // KC, (j % KC) // K1, j % K1
  W1_np = []
  for o in range(K1):
    idx = K1 * b + d - o
    valid = (idx >= 0) & (idx < K1)
    rows = np.clip(c_in * idx + c, 0, KC - 1)
    W1_np.append((rows, valid))
  W1 = jnp.concatenate(
      [jnp.where(jnp.asarray(v)[:, None], w1[jnp.asarray(r)], 0.0)
       for r, v in W1_np], axis=1).astype(jnp.bfloat16)

  def _shift_w(w, K, q, t0, t1, phase0, stride):
    wh = w.reshape(K * _CP, _CP)
    jj = np.arange(_CP * (t1 - t0))
    a, cc = t0 + jj // _CP, jj % _CP
    cols = []
    for h in range(2):
      k = stride * q + a - (phase0 + h)
      valid = (k >= 0) & (k < K)
      rows = np.clip(k * _CP + cc, 0, K * _CP - 1)
      cols.append(jnp.where(jnp.asarray(valid)[:, None],
                            wh[jnp.asarray(rows)], 0.0))
    return jnp.concatenate(cols, axis=1).astype(jnp.bfloat16)

  s2_plan = [[(0, 0, 4), (1, 0, 4), (2, 0, 1)],
             [(0, 2, 4), (1, 0, 4), (2, 0, 3)]]
  s3_plan = [(0, 0, 2), (1, 0, 2), (2, 0, 1)]
  W2s = [_shift_w(w2, K2, q, t0, t1, 2 * u, 4)
         for u in range(2) for (q, t0, t1) in s2_plan[u]]
  W3s = [_shift_w(w3, K3, q, t0, t1, 0, 2) for (q, t0, t1) in s3_plan]

  B = 4 if N % 4 == 0 else 1
  wspecs = [pl.BlockSpec(w.shape, lambda n: (0, 0)) for w in W2s + W3s]
  out = pl.pallas_call(
      lambda *refs: _body((B, BS, OP), s2_plan, s3_plan, *refs),
      out_shape=jax.ShapeDtypeStruct((N, 64, OP), jnp.bfloat16),
      grid=(N // B,),
      in_specs=[
          pl.BlockSpec((B, BSX, KC), lambda n: (n, 0, 0)),
          pl.BlockSpec(W1.shape, lambda n: (0, 0)),
          pl.BlockSpec(shifts.shape, lambda n: (0, 0)),
      ] + wspecs,
      out_specs=pl.BlockSpec((B, 64, OP), lambda n: (n, 0, 0)),
      scratch_shapes=[
          pltpu.VMEM((BS + 8, 4 * _CP), jnp.bfloat16),
          pltpu.VMEM((BS + 8, 2 * _CP), jnp.bfloat16),
      ],
      compiler_params=pltpu.CompilerParams(
          dimension_semantics=("parallel",)),
  )(xv, W1, shifts, *W2s, *W3s)

  return out[:, :50, :L_p3].astype(jnp.float32)


# speedup vs baseline: 1.6445x; 1.0596x over previous
import numpy as np

import jax
import jax.numpy as jnp
from jax.experimental import pallas as pl
from jax.experimental.pallas import tpu as pltpu

_CP = 128


def _round_up(x, m):
  return ((x + m - 1) // m) * m


def _body(dims, s2_plan, s3_plan, xv_ref, w1_ref, sh_ref, *rest):
  B, BS, OP = dims
  n2, n3 = len(s2_plan[0]) + len(s2_plan[1]), len(s3_plan)
  w2_refs = rest[:n2]
  w3_refs = rest[n2:n2 + n3]
  o_ref, pb, qb = rest[n2 + n3:]
  sh1 = sh_ref[0:1, :]
  sh2 = sh_ref[1:2, :]
  sh3 = sh_ref[2:3, :]

  for b in range(B):
    xvb = xv_ref[b]
    x2b = jnp.concatenate([xvb[0:BS], xvb[1:BS + 1]], axis=1)

    for u in range(4):
      y = jnp.dot(x2b, w1_ref[:, 2 * _CP * u:2 * _CP * (u + 1)],
                  preferred_element_type=jnp.float32)
      p = jnp.maximum(
          jnp.maximum(y[:, 0:_CP], y[:, _CP:2 * _CP]) + sh1,
          0.0).astype(jnp.bfloat16)
      pb[0:BS, _CP * u:_CP * (u + 1)] = p

    wi = 0
    for u in range(2):
      acc = None
      for q, t0, t1 in s2_plan[u]:
        d = jnp.dot(pb[q:q + BS, _CP * t0:_CP * t1], w2_refs[wi][...],
                    preferred_element_type=jnp.float32)
        acc = d if acc is None else acc + d
        wi += 1
      qv = jnp.maximum(
          jnp.maximum(acc[:, 0:_CP], acc[:, _CP:2 * _CP]) + sh2,
          0.0).astype(jnp.bfloat16)
      qb[0:BS, _CP * u:_CP * (u + 1)] = qv

    acc3 = None
    for wi3, (q, t0, t1) in enumerate(s3_plan):
      d = jnp.dot(qb[q:q + BS, _CP * t0:_CP * t1], w3_refs[wi3][...],
                  preferred_element_type=jnp.float32)
      acc3 = d if acc3 is None else acc3 + d
    fin = jnp.maximum(
        jnp.maximum(acc3[:, 0:_CP], acc3[:, _CP:2 * _CP]) + sh3, 0.0)
    fin_bf = fin.astype(jnp.bfloat16)
    ft = jnp.transpose(fin_bf[0:OP], (1, 0))
    o_ref[b] = ft[0:64, :]


@jax.jit
def kernel(x_ncl, w1, w2, w3, shifts):
  N, c_in, L = x_ncl.shape
  KC = w1.shape[0]
  K1 = KC // c_in
  K2, K3 = w2.shape[0], w3.shape[0]

  L0 = L + 8
  L_out1 = L0 - K1 + 1
  L_p1 = L_out1 // 2
  L_out2 = L_p1 - K2 + 1
  L_p2 = L_out2 // 2
  L_out3 = L_p2 - K3 + 1
  L_p3 = L_out3 // 2

  OP = _round_up(L_p3, 8)
  BS = _round_up(OP + 24, 16)
  BSX = BS + 8

  xp = jnp.pad(x_ncl, ((0, 0), (0, 0), (4, K1 * BSX - 4 - L)))
  xb = xp.astype(jnp.bfloat16).reshape(N, c_in, BSX, K1)
  xv = jnp.transpose(xb, (0, 2, 1, 3)).reshape(N, BSX, KC)

  j = np.arange(2 * KC)
  b, c, d = j // KC, (j % KC) // K1, j % K1
  W1_np = []
  for o in range(K1):
    idx = K1 * b + d - o
    valid = (idx >= 0) & (idx < K1)
    rows = np.clip(c_in * idx + c, 0, KC - 1)
    W1_np.append((rows, valid))
  W1 = jnp.concatenate(
      [jnp.where(jnp.asarray(v)[:, None], w1[jnp.asarray(r)], 0.0)
       for r, v in W1_np], axis=1).astype(jnp.bfloat16)

  def _shift_w(w, K, q, t0, t1, n_ph, stride):
    wh = w.reshape(K * _CP, _CP)
    jj = np.arange(_CP * (t1 - t0))
    a, cc = t0 + jj // _CP, jj % _CP
    cols = []
    for h in range(n_ph):
      k = stride * q + a - h
      valid = (k >= 0) & (k < K)
      rows = np.clip(k * _CP + cc, 0, K * _CP - 1)
      cols.append(jnp.where(jnp.asarray(valid)[:, None],
                            wh[jnp.asarray(rows)], 0.0))
    return jnp.concatenate(cols, axis=1).astype(jnp.bfloat16)

  s2_plan = [[(0, 0, 4), (1, 0, 4), (2, 0, 1)],
             [(0, 2, 4), (1, 0, 4), (2, 0, 3)]]
  s3_plan = [(0, 0, 2), (1, 0, 2), (2, 0, 1)]
  def _pair_w(w, K, q, t0, t1, u):
    wh = w.reshape(K * _CP, _CP)
    jj = np.arange(_CP * (t1 - t0))
    a, cc = t0 + jj // _CP, jj % _CP
    cols = []
    for h in range(2):
      k = 4 * q + a - (2 * u + h)
      valid = (k >= 0) & (k < K)
      rows = np.clip(k * _CP + cc, 0, K * _CP - 1)
      cols.append(jnp.where(jnp.asarray(valid)[:, None],
                            wh[jnp.asarray(rows)], 0.0))
    return jnp.concatenate(cols, axis=1).astype(jnp.bfloat16)

  W2s = [_pair_w(w2, K2, q, t0, t1, u)
         for u in range(2) for (q, t0, t1) in s2_plan[u]]
  W3s = [_shift_w(w3, K3, q, t0, t1, 2, 2) for (q, t0, t1) in s3_plan]

  B = 8 if N % 8 == 0 else 1
  wspecs = [pl.BlockSpec(w.shape, lambda n: (0, 0)) for w in W2s + W3s]
  out = pl.pallas_call(
      lambda *refs: _body((B, BS, OP), s2_plan, s3_plan, *refs),
      out_shape=jax.ShapeDtypeStruct((N, 64, OP), jnp.bfloat16),
      grid=(N // B,),
      in_specs=[
          pl.BlockSpec((B, BSX, KC), lambda n: (n, 0, 0)),
          pl.BlockSpec(W1.shape, lambda n: (0, 0)),
          pl.BlockSpec(shifts.shape, lambda n: (0, 0)),
      ] + wspecs,
      out_specs=pl.BlockSpec((B, 64, OP), lambda n: (n, 0, 0)),
      scratch_shapes=[
          pltpu.VMEM((BS + 8, 4 * _CP), jnp.bfloat16),
          pltpu.VMEM((BS + 8, 2 * _CP), jnp.bfloat16),
      ],
      compiler_params=pltpu.CompilerParams(
          dimension_semantics=("parallel",)),
  )(xv, W1, shifts, *W2s, *W3s)

  return out[:, :50, :L_p3].astype(jnp.float32)


# speedup vs baseline: 1.6775x; 1.0201x over previous
import numpy as np

import jax
import jax.numpy as jnp
from jax.experimental import pallas as pl
from jax.experimental.pallas import tpu as pltpu

_CP = 128


def _round_up(x, m):
  return ((x + m - 1) // m) * m


def _body(dims, s2_plan, s3_plan, xv_ref, w1_ref, sh_ref, *rest):
  B, BS, OP = dims
  n2, n3 = len(s2_plan[0]) + len(s2_plan[1]), len(s3_plan)
  w2_refs = rest[:n2]
  w3_refs = rest[n2:n2 + n3]
  o_ref, pb, qb = rest[n2 + n3:]
  sh1 = sh_ref[0:1, :]
  sh2 = sh_ref[1:2, :]
  sh3 = sh_ref[2:3, :]

  for b in range(B):
    xvb = xv_ref[b]
    x2b = jnp.concatenate([xvb[0:BS], xvb[1:BS + 1]], axis=1)

    for u in range(4):
      y = jnp.dot(x2b, w1_ref[:, 2 * _CP * u:2 * _CP * (u + 1)],
                  preferred_element_type=jnp.float32)
      p = jnp.maximum(
          jnp.maximum(y[:, 0:_CP], y[:, _CP:2 * _CP]) + sh1,
          0.0).astype(jnp.bfloat16)
      pb[0:BS, _CP * u:_CP * (u + 1)] = p

    wi = 0
    for u in range(2):
      acc = None
      for q, t0, t1 in s2_plan[u]:
        d = jnp.dot(pb[q:q + BS, _CP * t0:_CP * t1], w2_refs[wi][...],
                    preferred_element_type=jnp.float32)
        acc = d if acc is None else acc + d
        wi += 1
      qv = jnp.maximum(
          jnp.maximum(acc[:, 0:_CP], acc[:, _CP:2 * _CP]) + sh2,
          0.0).astype(jnp.bfloat16)
      qb[0:BS, _CP * u:_CP * (u + 1)] = qv

    acc3 = None
    for wi3, (q, t0, t1) in enumerate(s3_plan):
      d = jnp.dot(qb[q:q + BS, _CP * t0:_CP * t1], w3_refs[wi3][...],
                  preferred_element_type=jnp.float32)
      acc3 = d if acc3 is None else acc3 + d
    fin = jnp.maximum(
        jnp.maximum(acc3[:, 0:_CP], acc3[:, _CP:2 * _CP]) + sh3, 0.0)
    fin_bf = fin.astype(jnp.bfloat16)
    ft = jnp.transpose(fin_bf[0:OP], (1, 0))
    o_ref[b] = ft[0:64, :]


@jax.jit
def kernel(x_ncl, w1, w2, w3, shifts):
  N, c_in, L = x_ncl.shape
  KC = w1.shape[0]
  K1 = KC // c_in
  K2, K3 = w2.shape[0], w3.shape[0]

  L0 = L + 8
  L_out1 = L0 - K1 + 1
  L_p1 = L_out1 // 2
  L_out2 = L_p1 - K2 + 1
  L_p2 = L_out2 // 2
  L_out3 = L_p2 - K3 + 1
  L_p3 = L_out3 // 2

  OP = _round_up(L_p3, 8)
  BS = _round_up(OP + 24, 16)
  BSX = BS + 8

  xp = jnp.pad(x_ncl, ((0, 0), (0, 0), (4, K1 * BSX - 4 - L)))
  xb = xp.astype(jnp.bfloat16).reshape(N, c_in, BSX, K1)
  xv = jnp.transpose(xb, (0, 2, 1, 3)).reshape(N, BSX, KC)

  j = np.arange(2 * KC)
  b, c, d = j // KC, (j % KC) // K1, j % K1
  W1_np = []
  for o in range(K1):
    idx = K1 * b + d - o
    valid = (idx >= 0) & (idx < K1)
    rows = np.clip(c_in * idx + c, 0, KC - 1)
    W1_np.append((rows, valid))
  W1 = jnp.concatenate(
      [jnp.where(jnp.asarray(v)[:, None], w1[jnp.asarray(r)], 0.0)
       for r, v in W1_np], axis=1).astype(jnp.bfloat16)

  def _shift_w(w, K, q, t0, t1, n_ph, stride):
    wh = w.reshape(K * _CP, _CP)
    jj = np.arange(_CP * (t1 - t0))
    a, cc = t0 + jj // _CP, jj % _CP
    cols = []
    for h in range(n_ph):
      k = stride * q + a - h
      valid = (k >= 0) & (k < K)
      rows = np.clip(k * _CP + cc, 0, K * _CP - 1)
      cols.append(jnp.where(jnp.asarray(valid)[:, None],
                            wh[jnp.asarray(rows)], 0.0))
    return jnp.concatenate(cols, axis=1).astype(jnp.bfloat16)

  s2_plan = [[(0, 0, 4), (1, 0, 4), (2, 0, 1)],
             [(0, 2, 4), (1, 0, 4), (2, 0, 3)]]
  s3_plan = [(0, 0, 2), (1, 0, 2), (2, 0, 1)]
  def _pair_w(w, K, q, t0, t1, u):
    wh = w.reshape(K * _CP, _CP)
    jj = np.arange(_CP * (t1 - t0))
    a, cc = t0 + jj // _CP, jj % _CP
    cols = []
    for h in range(2):
      k = 4 * q + a - (2 * u + h)
      valid = (k >= 0) & (k < K)
      rows = np.clip(k * _CP + cc, 0, K * _CP - 1)
      cols.append(jnp.where(jnp.asarray(valid)[:, None],
                            wh[jnp.asarray(rows)], 0.0))
    return jnp.concatenate(cols, axis=1).astype(jnp.bfloat16)

  W2s = [_pair_w(w2, K2, q, t0, t1, u)
         for u in range(2) for (q, t0, t1) in s2_plan[u]]
  W3s = [_shift_w(w3, K3, q, t0, t1, 2, 2) for (q, t0, t1) in s3_plan]

  B = 16 if N % 16 == 0 else 1
  wspecs = [pl.BlockSpec(w.shape, lambda n: (0, 0)) for w in W2s + W3s]
  out = pl.pallas_call(
      lambda *refs: _body((B, BS, OP), s2_plan, s3_plan, *refs),
      out_shape=jax.ShapeDtypeStruct((N, 64, OP), jnp.bfloat16),
      grid=(N // B,),
      in_specs=[
          pl.BlockSpec((B, BSX, KC), lambda n: (n, 0, 0)),
          pl.BlockSpec(W1.shape, lambda n: (0, 0)),
          pl.BlockSpec(shifts.shape, lambda n: (0, 0)),
      ] + wspecs,
      out_specs=pl.BlockSpec((B, 64, OP), lambda n: (n, 0, 0)),
      scratch_shapes=[
          pltpu.VMEM((BS + 8, 4 * _CP), jnp.bfloat16),
          pltpu.VMEM((BS + 8, 2 * _CP), jnp.bfloat16),
      ],
      compiler_params=pltpu.CompilerParams(
          dimension_semantics=("parallel",)),
  )(xv, W1, shifts, *W2s, *W3s)

  return out[:, :50, :L_p3].astype(jnp.float32)


# speedup vs baseline: 1.7301x; 1.0314x over previous
import numpy as np

import jax
import jax.numpy as jnp
from jax.experimental import pallas as pl
from jax.experimental.pallas import tpu as pltpu

_CP = 128


def _round_up(x, m):
  return ((x + m - 1) // m) * m


def _body(dims, s2_plan, s3_plan, xv_ref, w1_ref, sh_ref, *rest):
  B, BS, OP = dims
  n2, n3 = len(s2_plan[0]) + len(s2_plan[1]), len(s3_plan)
  w2_refs = rest[:n2]
  w3_refs = rest[n2:n2 + n3]
  o_ref, pb, qb = rest[n2 + n3:]
  sh1 = sh_ref[0:1, :]
  sh2 = sh_ref[1:2, :]
  sh3 = sh_ref[2:3, :]

  for b in range(B):
    xvb = xv_ref[b]
    x2b = jnp.concatenate([xvb[0:BS], xvb[1:BS + 1]], axis=1)

    for u in range(4):
      y = jnp.dot(x2b, w1_ref[:, 2 * _CP * u:2 * _CP * (u + 1)],
                  preferred_element_type=jnp.float32)
      p = jnp.maximum(
          jnp.maximum(y[:, 0:_CP], y[:, _CP:2 * _CP]) + sh1,
          0.0).astype(jnp.bfloat16)
      pb[0:BS, _CP * u:_CP * (u + 1)] = p

    wi = 0
    for u in range(2):
      acc = None
      for q, t0, t1 in s2_plan[u]:
        d = jnp.dot(pb[q:q + BS, _CP * t0:_CP * t1], w2_refs[wi][...],
                    preferred_element_type=jnp.float32)
        acc = d if acc is None else acc + d
        wi += 1
      qv = jnp.maximum(
          jnp.maximum(acc[:, 0:_CP], acc[:, _CP:2 * _CP]) + sh2,
          0.0).astype(jnp.bfloat16)
      qb[0:BS, _CP * u:_CP * (u + 1)] = qv

    acc3 = None
    for wi3, (q, t0, t1) in enumerate(s3_plan):
      d = jnp.dot(qb[q:q + BS, _CP * t0:_CP * t1], w3_refs[wi3][...],
                  preferred_element_type=jnp.float32)
      acc3 = d if acc3 is None else acc3 + d
    fin = jnp.maximum(
        jnp.maximum(acc3[:, 0:_CP], acc3[:, _CP:2 * _CP]) + sh3, 0.0)
    ft = jnp.transpose(fin[0:OP], (1, 0))
    o_ref[b] = ft[0:o_ref.shape[1], 0:o_ref.shape[2]]


@jax.jit
def kernel(x_ncl, w1, w2, w3, shifts):
  N, c_in, L = x_ncl.shape
  KC = w1.shape[0]
  K1 = KC // c_in
  K2, K3 = w2.shape[0], w3.shape[0]

  L0 = L + 8
  L_out1 = L0 - K1 + 1
  L_p1 = L_out1 // 2
  L_out2 = L_p1 - K2 + 1
  L_p2 = L_out2 // 2
  L_out3 = L_p2 - K3 + 1
  L_p3 = L_out3 // 2

  OP = _round_up(L_p3, 8)
  BS = _round_up(OP + 24, 16)
  BSX = BS + 8

  xp = jnp.pad(x_ncl, ((0, 0), (0, 0), (4, K1 * BSX - 4 - L)))
  xb = xp.astype(jnp.bfloat16).reshape(N, c_in, BSX, K1)
  xv = jnp.transpose(xb, (0, 2, 1, 3)).reshape(N, BSX, KC)

  j = np.arange(2 * KC)
  b, c, d = j // KC, (j % KC) // K1, j % K1
  W1_np = []
  for o in range(K1):
    idx = K1 * b + d - o
    valid = (idx >= 0) & (idx < K1)
    rows = np.clip(c_in * idx + c, 0, KC - 1)
    W1_np.append((rows, valid))
  W1 = jnp.concatenate(
      [jnp.where(jnp.asarray(v)[:, None], w1[jnp.asarray(r)], 0.0)
       for r, v in W1_np], axis=1).astype(jnp.bfloat16)

  def _shift_w(w, K, q, t0, t1, n_ph, stride):
    wh = w.reshape(K * _CP, _CP)
    jj = np.arange(_CP * (t1 - t0))
    a, cc = t0 + jj // _CP, jj % _CP
    cols = []
    for h in range(n_ph):
      k = stride * q + a - h
      valid = (k >= 0) & (k < K)
      rows = np.clip(k * _CP + cc, 0, K * _CP - 1)
      cols.append(jnp.where(jnp.asarray(valid)[:, None],
                            wh[jnp.asarray(rows)], 0.0))
    return jnp.concatenate(cols, axis=1).astype(jnp.bfloat16)

  s2_plan = [[(0, 0, 4), (1, 0, 4), (2, 0, 1)],
             [(0, 2, 4), (1, 0, 4), (2, 0, 3)]]
  s3_plan = [(0, 0, 2), (1, 0, 2), (2, 0, 1)]
  def _pair_w(w, K, q, t0, t1, u):
    wh = w.reshape(K * _CP, _CP)
    jj = np.arange(_CP * (t1 - t0))
    a, cc = t0 + jj // _CP, jj % _CP
    cols = []
    for h in range(2):
      k = 4 * q + a - (2 * u + h)
      valid = (k >= 0) & (k < K)
      rows = np.clip(k * _CP + cc, 0, K * _CP - 1)
      cols.append(jnp.where(jnp.asarray(valid)[:, None],
                            wh[jnp.asarray(rows)], 0.0))
    return jnp.concatenate(cols, axis=1).astype(jnp.bfloat16)

  W2s = [_pair_w(w2, K2, q, t0, t1, u)
         for u in range(2) for (q, t0, t1) in s2_plan[u]]
  W3s = [_shift_w(w3, K3, q, t0, t1, 2, 2) for (q, t0, t1) in s3_plan]

  B = 16 if N % 16 == 0 else 1
  wspecs = [pl.BlockSpec(w.shape, lambda n: (0, 0)) for w in W2s + W3s]
  out = pl.pallas_call(
      lambda *refs: _body((B, BS, OP), s2_plan, s3_plan, *refs),
      out_shape=jax.ShapeDtypeStruct((N, 50, L_p3), jnp.float32),
      grid=(N // B,),
      in_specs=[
          pl.BlockSpec((B, BSX, KC), lambda n: (n, 0, 0)),
          pl.BlockSpec(W1.shape, lambda n: (0, 0)),
          pl.BlockSpec(shifts.shape, lambda n: (0, 0)),
      ] + wspecs,
      out_specs=pl.BlockSpec((B, 50, L_p3), lambda n: (n, 0, 0)),
      scratch_shapes=[
          pltpu.VMEM((BS + 8, 4 * _CP), jnp.bfloat16),
          pltpu.VMEM((BS + 8, 2 * _CP), jnp.bfloat16),
      ],
      compiler_params=pltpu.CompilerParams(
          dimension_semantics=("parallel",)),
  )(xv, W1, shifts, *W2s, *W3s)

  return out
